# Initial kernel scaffold; baseline (speedup 1.0000x reference)
import jax
import jax.numpy as jnp
from jax.experimental import pallas as pl


def kernel(x, edge_index, W1, b1, W2, b2, W3, b3):
    raise NotImplementedError("write your pallas kernel here")


# speedup vs baseline: 10.6218x; 10.6218x over previous
import functools

import jax
import jax.numpy as jnp
from jax import lax
from jax.experimental import pallas as pl
from jax.experimental.pallas import tpu as pltpu
from jax.experimental.pallas import tpu_sc as plsc

N = 10000
E = 320000
D = 128
NC, NS = 2, 16
NW = NC * NS
CHUNK = 128
CPT = 79
EPT = CHUNK * CPT
E_PAD = NW * EPT
RPT = 632
ACC_ROWS = NS * RPT
DUMP = N
DEG_W = 8
BLK = 400


def _make_sc_scatter():
    mesh = plsc.VectorSubcoreMesh(core_axis_name="c", subcore_axis_name="s")

    @functools.partial(
        pl.kernel,
        mesh=mesh,
        out_type=jax.ShapeDtypeStruct((NC, ACC_ROWS, D), jnp.float32),
        scratch_types=[
            pltpu.VMEM((CPT, CHUNK), jnp.int32),
            pltpu.VMEM((CPT, CHUNK), jnp.int32),
            pltpu.VMEM((CHUNK, D), jnp.float32),
            pltpu.VMEM_SHARED((ACC_ROWS, D), jnp.float32),
        ],
    )
    def sc_scatter(g_hbm, src_hbm, dst_hbm, zrows_hbm, out_hbm,
                   src_v, dst_v, rows_v, acc):
        c = lax.axis_index("c")
        s = lax.axis_index("s")
        wid = c * NS + s
        pltpu.sync_copy(src_hbm.at[wid], src_v)
        pltpu.sync_copy(dst_hbm.at[wid], dst_v)
        base = s * RPT
        pltpu.sync_copy(zrows_hbm, acc.at[pl.ds(base, RPT)])
        plsc.subcore_barrier()

        def body(j, carry):
            pltpu.sync_copy(g_hbm.at[src_v.at[j]], rows_v)
            pltpu.sync_copy(rows_v, acc.at[dst_v.at[j]], add=True)
            return carry

        lax.fori_loop(0, CPT, body, 0)
        plsc.subcore_barrier()
        pltpu.sync_copy(acc.at[pl.ds(base, RPT)], out_hbm.at[c, pl.ds(base, RPT)])

    return sc_scatter


def _make_sc_degree():
    mesh = plsc.VectorSubcoreMesh(core_axis_name="c", subcore_axis_name="s")

    @functools.partial(
        pl.kernel,
        mesh=mesh,
        out_type=jax.ShapeDtypeStruct((NC, ACC_ROWS, D), jnp.float32),
        scratch_types=[
            pltpu.VMEM((CPT, CHUNK), jnp.int32),
            pltpu.VMEM((CHUNK, D), jnp.float32),
            pltpu.VMEM_SHARED((ACC_ROWS, D), jnp.float32),
        ],
    )
    def sc_degree(dst_hbm, ones_hbm, zrows_hbm, out_hbm, dst_v, ones_v, acc):
        c = lax.axis_index("c")
        s = lax.axis_index("s")
        wid = c * NS + s
        pltpu.sync_copy(dst_hbm.at[wid], dst_v)
        pltpu.sync_copy(ones_hbm, ones_v)
        base = s * RPT
        pltpu.sync_copy(zrows_hbm, acc.at[pl.ds(base, RPT)])
        plsc.subcore_barrier()

        def body(j, carry):
            pltpu.sync_copy(ones_v, acc.at[dst_v.at[j]], add=True)
            return carry

        lax.fori_loop(0, CPT, body, 0)
        plsc.subcore_barrier()
        pltpu.sync_copy(acc.at[pl.ds(base, RPT)], out_hbm.at[c, pl.ds(base, RPT)])

    return sc_degree


_sc_scatter = _make_sc_scatter()
_sc_degree = _make_sc_degree()


def _row_spec():
    return pl.BlockSpec((BLK, D), lambda i: (i, 0))


def _deg_spec():
    return pl.BlockSpec((BLK, DEG_W), lambda i: (i, 0))


def _full_spec(shape):
    return pl.BlockSpec(shape, lambda i: (0,) * len(shape))


def _dinv(dp0_ref, dp1_ref):
    deg = dp0_ref[:, 0:1] + dp1_ref[:, 0:1] + 1.0
    return lax.rsqrt(deg)


def _tc_matmul_body(x, w, xw_out):
    xw_out[...] = jnp.dot(x[...], w[...], preferred_element_type=jnp.float32)


def _tc_scale_body(dp0, dp1, xw, g_out):
    g_out[...] = _dinv(dp0, dp1) * xw[...]


def _tc_mid_body(dp0, dp1, s0, s1, g, b, w, g_out):
    dinv = _dinv(dp0, dp1)
    h = dinv * (s0[...] + s1[...] + g[...]) + b[...]
    h = jax.nn.gelu(h)
    g_out[...] = dinv * jnp.dot(h, w[...], preferred_element_type=jnp.float32)


def _tc_last_body(dp0, dp1, s0, s1, g, b, out):
    dinv = _dinv(dp0, dp1)
    out[...] = dinv * (s0[...] + s1[...] + g[...]) + b[...]


_GRID = (N // BLK,)
_OUT = jax.ShapeDtypeStruct((N, D), jnp.float32)

_tc_matmul = pl.pallas_call(
    _tc_matmul_body,
    grid=_GRID,
    in_specs=[_row_spec(), _full_spec((D, D))],
    out_specs=_row_spec(),
    out_shape=_OUT,
)

_tc_scale = pl.pallas_call(
    _tc_scale_body,
    grid=_GRID,
    in_specs=[_deg_spec(), _deg_spec(), _row_spec()],
    out_specs=_row_spec(),
    out_shape=_OUT,
)

_tc_mid = pl.pallas_call(
    _tc_mid_body,
    grid=_GRID,
    in_specs=[_deg_spec(), _deg_spec(), _row_spec(), _row_spec(), _row_spec(),
              _full_spec((1, D)), _full_spec((D, D))],
    out_specs=_row_spec(),
    out_shape=_OUT,
)

_tc_last = pl.pallas_call(
    _tc_last_body,
    grid=_GRID,
    in_specs=[_deg_spec(), _deg_spec(), _row_spec(), _row_spec(), _row_spec(),
              _full_spec((1, D))],
    out_specs=_row_spec(),
    out_shape=_OUT,
)


def kernel(x, edge_index, W1, b1, W2, b2, W3, b3):
    src = edge_index[0]
    dst = edge_index[1]
    pad = E_PAD - E
    src3 = jnp.concatenate([src, jnp.zeros((pad,), jnp.int32)]).reshape(NW, CPT, CHUNK)
    dst3 = jnp.concatenate([dst, jnp.full((pad,), DUMP, jnp.int32)]).reshape(NW, CPT, CHUNK)
    zrows = jnp.zeros((RPT, D), jnp.float32)
    ones_rows = jnp.ones((CHUNK, D), jnp.float32)

    degp = _sc_degree(dst3, ones_rows, zrows)
    xw1 = _tc_matmul(x, W1)
    dp0 = degp[0, :N, :DEG_W]
    dp1 = degp[1, :N, :DEG_W]
    b1r, b2r, b3r = (b.reshape(1, D) for b in (b1, b2, b3))

    g1 = _tc_scale(dp0, dp1, xw1)
    S1 = _sc_scatter(g1, src3, dst3, zrows)
    g2 = _tc_mid(dp0, dp1, S1[0, :N], S1[1, :N], g1, b1r, W2)
    S2 = _sc_scatter(g2, src3, dst3, zrows)
    g3 = _tc_mid(dp0, dp1, S2[0, :N], S2[1, :N], g2, b2r, W3)
    S3 = _sc_scatter(g3, src3, dst3, zrows)
    return _tc_last(dp0, dp1, S3[0, :N], S3[1, :N], g3, b3r)
